# Initial kernel scaffold; baseline (speedup 1.0000x reference)
import jax
import jax.numpy as jnp
from jax.experimental import pallas as pl


def kernel(x_user, x_item, edge_index_user_to_item, edge_index_item_rev_user, Wl0_u2i, Wr0_u2i, att0_u2i, b0_u2i, Wl0_i2u, Wr0_i2u, att0_i2u, b0_i2u, Wl1_u2i, Wr1_u2i, att1_u2i, b1_u2i, Wl1_i2u, Wr1_i2u, att1_i2u, b1_i2u):
    raise NotImplementedError("write your pallas kernel here")


# speedup vs baseline: 7.5121x; 7.5121x over previous
import functools

import jax
import jax.numpy as jnp
from jax import lax
from jax.experimental import pallas as pl
from jax.experimental.pallas import tpu as pltpu
from jax.experimental.pallas import tpu_sc as plsc

NC = 2
NS = 16
LANES = 16
DPAD = 16


def _proj2_body(x_ref, w1_ref, w2_ref, o1_ref, o2_ref):
    x = x_ref[...]
    o1_ref[...] = jnp.dot(x, w1_ref[...], preferred_element_type=jnp.float32)
    o2_ref[...] = jnp.dot(x, w2_ref[...], preferred_element_type=jnp.float32)


def _proj2(x, w1, w2):
    n, k = x.shape
    blk = 2000
    return pl.pallas_call(
        _proj2_body,
        grid=(n // blk,),
        in_specs=[pl.BlockSpec((blk, k), lambda i: (i, 0)),
                  pl.BlockSpec((k, w1.shape[1]), lambda i: (0, 0)),
                  pl.BlockSpec((k, w2.shape[1]), lambda i: (0, 0))],
        out_specs=[pl.BlockSpec((blk, w1.shape[1]), lambda i: (i, 0)),
                   pl.BlockSpec((blk, w2.shape[1]), lambda i: (i, 0))],
        out_shape=[jax.ShapeDtypeStruct((n, w1.shape[1]), jnp.float32),
                   jax.ShapeDtypeStruct((n, w2.shape[1]), jnp.float32)],
    )(x, w1, w2)


def _head_select(heads, ch, f):
    r = lax.broadcasted_iota(jnp.int32, (DPAD, f), 0)
    c = lax.broadcasted_iota(jnp.int32, (DPAD, f), 1) // ch
    return (r == c).astype(jnp.float32)


def _combo_body(heads, ch, op_ref, dp_ref, b_ref, w1_ref, w2_ref,
                o1_ref, o2_ref):
    f = heads * ch
    o = op_ref[0] + op_ref[1]
    d = dp_ref[0] + dp_ref[1]
    db = jnp.dot(d, _head_select(heads, ch, f),
                 preferred_element_type=jnp.float32)
    h = o / (db + 1e-16) + b_ref[...]
    h = jnp.maximum(h, 0.0)
    o1_ref[...] = jnp.dot(h, w1_ref[...], preferred_element_type=jnp.float32)
    o2_ref[...] = jnp.dot(h, w2_ref[...], preferred_element_type=jnp.float32)


def _combo(op, dp, b, w1, w2, heads, ch):
    n = op.shape[1]
    f = heads * ch
    m1, m2 = w1.shape[1], w2.shape[1]
    blk = 2000
    b2 = b.reshape(1, f)
    return pl.pallas_call(
        functools.partial(_combo_body, heads, ch),
        grid=(n // blk,),
        in_specs=[pl.BlockSpec((2, blk, f), lambda i: (0, i, 0)),
                  pl.BlockSpec((2, blk, DPAD), lambda i: (0, i, 0)),
                  pl.BlockSpec((1, f), lambda i: (0, 0)),
                  pl.BlockSpec((f, m1), lambda i: (0, 0)),
                  pl.BlockSpec((f, m2), lambda i: (0, 0))],
        out_specs=[pl.BlockSpec((blk, m1), lambda i: (i, 0)),
                   pl.BlockSpec((blk, m2), lambda i: (i, 0))],
        out_shape=[jax.ShapeDtypeStruct((n, m1), jnp.float32),
                   jax.ShapeDtypeStruct((n, m2), jnp.float32)],
    )(op, dp, b2, w1, w2)


def _final_body(heads, ch, op_ref, dp_ref, b_ref, o_ref):
    f = heads * ch
    o = op_ref[0] + op_ref[1]
    d = dp_ref[0] + dp_ref[1]
    db = jnp.dot(d, _head_select(heads, ch, f),
                 preferred_element_type=jnp.float32)
    o_ref[...] = o / (db + 1e-16) + b_ref[...]


def _final(op, dp, b, heads, ch):
    n = op.shape[1]
    f = heads * ch
    blk = 2000
    b2 = b.reshape(1, f)
    return pl.pallas_call(
        functools.partial(_final_body, heads, ch),
        grid=(n // blk,),
        in_specs=[pl.BlockSpec((2, blk, f), lambda i: (0, i, 0)),
                  pl.BlockSpec((2, blk, DPAD), lambda i: (0, i, 0)),
                  pl.BlockSpec((1, f), lambda i: (0, 0))],
        out_specs=pl.BlockSpec((blk, f), lambda i: (i, 0)),
        out_shape=jax.ShapeDtypeStruct((n, f), jnp.float32),
    )(op, dp, b2)


def _edge_sc(xl, xr, src, dst, att, n_dst, heads, ch):
    e = src.shape[0]
    f = heads * ch
    chunk = 80 if f >= 128 else 320
    epw = e // (NC * NS)
    nch = epw // chunk
    ndp = -(-n_dst // (8 * NS)) * (8 * NS)
    rpt = ndp // NS
    rz = 8
    mesh = plsc.VectorSubcoreMesh(core_axis_name="c", subcore_axis_name="s",
                                  num_cores=NC, num_subcores=NS)

    def body(xl_hbm, xr_hbm, src_hbm, dst_hbm, att_hbm, out_hbm, den_hbm,
             src_idx, dst_idx, xl_buf, xr_buf, den_buf, att_buf,
             out_sh, den_sh, sem):
        c = lax.axis_index("c")
        s = lax.axis_index("s")
        wid = c * NS + s
        pltpu.sync_copy(att_hbm, att_buf)

        def zrow(i, carry):
            for j in range(f // LANES):
                xl_buf[i, pl.ds(j * LANES, LANES)] = jnp.zeros(
                    (LANES,), jnp.float32)
            den_buf[i, :] = jnp.zeros((LANES,), jnp.float32)
            return carry
        lax.fori_loop(0, chunk, zrow, 0)
        rbase = pl.multiple_of(s * rpt, 8)

        def zfill(i, carry):
            ro = pl.multiple_of(rbase + i * rz, 8)
            pltpu.sync_copy(xl_buf.at[pl.ds(0, rz)], out_sh.at[pl.ds(ro, rz)])
            pltpu.sync_copy(den_buf.at[pl.ds(0, rz)], den_sh.at[pl.ds(ro, rz)])
            return carry
        lax.fori_loop(0, rpt // rz, zfill, 0)
        plsc.subcore_barrier()

        lane = lax.iota(jnp.int32, LANES)

        def group_body(g, carry):
            rvec = g * LANES + lane
            acc = [jnp.zeros((LANES,), jnp.float32) for _ in range(heads)]
            for col in range(f):
                cv = jnp.full((LANES,), col, jnp.int32)
                a = plsc.load_gather(xl_buf, [rvec, cv])
                b = plsc.load_gather(xr_buf, [rvec, cv])
                u = a + b
                u = jnp.maximum(u, 0.2 * u)
                h = col // ch
                acc[h] = acc[h] + u * att_buf[col, :]
            pe = [jnp.exp(acc[h]) for h in range(heads)]
            for h in range(heads):
                hv = jnp.full((LANES,), h, jnp.int32)
                plsc.store_scatter(den_buf, [rvec, hv], pe[h])
            for col in range(f):
                cv = jnp.full((LANES,), col, jnp.int32)
                a = plsc.load_gather(xl_buf, [rvec, cv])
                plsc.store_scatter(xl_buf, [rvec, cv], a * pe[col // ch])
            return carry

        ebase = wid * epw

        def chunk_body(g, carry):
            base = pl.multiple_of(ebase + g * chunk, 8)
            pltpu.sync_copy(src_hbm.at[pl.ds(base, chunk)], src_idx)
            pltpu.sync_copy(dst_hbm.at[pl.ds(base, chunk)], dst_idx)
            pltpu.async_copy(xl_hbm.at[src_idx], xl_buf, sem).wait()
            pltpu.async_copy(xr_hbm.at[dst_idx], xr_buf, sem).wait()
            lax.fori_loop(0, chunk // LANES, group_body, 0)
            pltpu.sync_copy(xl_buf, out_sh.at[dst_idx], add=True)
            pltpu.sync_copy(den_buf, den_sh.at[dst_idx], add=True)
            return carry

        lax.fori_loop(0, nch, chunk_body, 0)
        plsc.subcore_barrier()
        pltpu.sync_copy(out_sh.at[pl.ds(rbase, rpt)],
                        out_hbm.at[c, pl.ds(rbase, rpt)])
        pltpu.sync_copy(den_sh.at[pl.ds(rbase, rpt)],
                        den_hbm.at[c, pl.ds(rbase, rpt)])

    run = pl.kernel(
        body,
        out_type=[jax.ShapeDtypeStruct((NC, ndp, f), jnp.float32),
                  jax.ShapeDtypeStruct((NC, ndp, DPAD), jnp.float32)],
        mesh=mesh,
        compiler_params=pltpu.CompilerParams(needs_layout_passes=False,
                                             use_tc_tiling_on_sc=False),
        scratch_types=[
            pltpu.VMEM((chunk,), jnp.int32),
            pltpu.VMEM((chunk,), jnp.int32),
            pltpu.VMEM((chunk, f), jnp.float32),
            pltpu.VMEM((chunk, f), jnp.float32),
            pltpu.VMEM((chunk, DPAD), jnp.float32),
            pltpu.VMEM((f, LANES), jnp.float32),
            pltpu.VMEM_SHARED((ndp, f), jnp.float32),
            pltpu.VMEM_SHARED((ndp, DPAD), jnp.float32),
            pltpu.SemaphoreType.DMA,
        ],
    )
    att_rep = jnp.tile(att.reshape(-1, 1), (1, LANES))
    op, dp = run(xl, xr, src, dst, att_rep)
    return op[:, :n_dst], dp[:, :n_dst]


def kernel(x_user, x_item, edge_index_user_to_item, edge_index_item_rev_user,
           Wl0_u2i, Wr0_u2i, att0_u2i, b0_u2i,
           Wl0_i2u, Wr0_i2u, att0_i2u, b0_i2u,
           Wl1_u2i, Wr1_u2i, att1_u2i, b1_u2i,
           Wl1_i2u, Wr1_i2u, att1_i2u, b1_i2u):
    n_user = x_user.shape[0]
    n_item = x_item.shape[0]
    heads0, ch0 = att0_u2i.shape
    out1 = att1_u2i.shape[1]

    ne = edge_index_user_to_item.shape[1]
    nep = -(-ne // (NC * NS * 320)) * (NC * NS * 320)
    pad_s = jnp.zeros((nep - ne,), edge_index_user_to_item.dtype)
    pad_d = jnp.full((nep - ne,), n_item, edge_index_user_to_item.dtype)
    src_u2i = jnp.concatenate([edge_index_user_to_item[0], pad_s])
    dst_u2i = jnp.concatenate([edge_index_user_to_item[1], pad_d])
    src_i2u = jnp.concatenate([edge_index_item_rev_user[0], pad_s])
    dst_i2u = jnp.concatenate([edge_index_item_rev_user[1],
                               jnp.full((nep - ne,), n_user,
                                        edge_index_item_rev_user.dtype)])

    xl0_u2i, xr0_i2u = _proj2(x_user, Wl0_u2i, Wr0_i2u)
    xl0_i2u, xr0_u2i = _proj2(x_item, Wl0_i2u, Wr0_u2i)

    op_i0, dp_i0 = _edge_sc(xl0_u2i, xr0_u2i, src_u2i, dst_u2i,
                            att0_u2i.reshape(-1), n_item, heads0, ch0)
    op_u0, dp_u0 = _edge_sc(xl0_i2u, xr0_i2u, src_i2u, dst_i2u,
                            att0_i2u.reshape(-1), n_user, heads0, ch0)

    xl1_u2i, xr1_i2u = _combo(op_u0, dp_u0, b0_i2u, Wl1_u2i, Wr1_i2u,
                              heads0, ch0)
    xl1_i2u, xr1_u2i = _combo(op_i0, dp_i0, b0_u2i, Wl1_i2u, Wr1_u2i,
                              heads0, ch0)

    op_i1, dp_i1 = _edge_sc(xl1_u2i, xr1_u2i, src_u2i, dst_u2i,
                            att1_u2i.reshape(-1), n_item, 1, out1)
    op_u1, dp_u1 = _edge_sc(xl1_i2u, xr1_i2u, src_i2u, dst_i2u,
                            att1_i2u.reshape(-1), n_user, 1, out1)

    out_user = _final(op_u1, dp_u1, b1_i2u, 1, out1)
    out_item = _final(op_i1, dp_i1, b1_u2i, 1, out1)
    return (out_user, out_item)


# speedup vs baseline: 9.5866x; 1.2762x over previous
import functools

import jax
import jax.numpy as jnp
from jax import lax
from jax.experimental import pallas as pl
from jax.experimental.pallas import tpu as pltpu
from jax.experimental.pallas import tpu_sc as plsc

NC = 2
NS = 16
LANES = 16
DPAD = 16


def _proj2_body(x_ref, w1_ref, w2_ref, o1_ref, o2_ref):
    x = x_ref[...]
    o1_ref[...] = jnp.dot(x, w1_ref[...], preferred_element_type=jnp.float32)
    o2_ref[...] = jnp.dot(x, w2_ref[...], preferred_element_type=jnp.float32)


def _proj2(x, w1, w2):
    n, k = x.shape
    blk = 2000
    return pl.pallas_call(
        _proj2_body,
        grid=(n // blk,),
        in_specs=[pl.BlockSpec((blk, k), lambda i: (i, 0)),
                  pl.BlockSpec((k, w1.shape[1]), lambda i: (0, 0)),
                  pl.BlockSpec((k, w2.shape[1]), lambda i: (0, 0))],
        out_specs=[pl.BlockSpec((blk, w1.shape[1]), lambda i: (i, 0)),
                   pl.BlockSpec((blk, w2.shape[1]), lambda i: (i, 0))],
        out_shape=[jax.ShapeDtypeStruct((n, w1.shape[1]), jnp.float32),
                   jax.ShapeDtypeStruct((n, w2.shape[1]), jnp.float32)],
    )(x, w1, w2)


def _head_select(heads, ch, f):
    r = lax.broadcasted_iota(jnp.int32, (DPAD, f), 0)
    c = lax.broadcasted_iota(jnp.int32, (DPAD, f), 1) // ch
    return (r == c).astype(jnp.float32)


def _combo_body(heads, ch, op_ref, dp_ref, b_ref, w1_ref, w2_ref,
                o1_ref, o2_ref):
    f = heads * ch
    o = op_ref[0] + op_ref[1]
    d = dp_ref[0] + dp_ref[1]
    db = jnp.dot(d, _head_select(heads, ch, f),
                 preferred_element_type=jnp.float32)
    h = o / (db + 1e-16) + b_ref[...]
    h = jnp.maximum(h, 0.0)
    o1_ref[...] = jnp.dot(h, w1_ref[...], preferred_element_type=jnp.float32)
    o2_ref[...] = jnp.dot(h, w2_ref[...], preferred_element_type=jnp.float32)


def _combo(op, dp, b, w1, w2, heads, ch):
    n = op.shape[1]
    f = heads * ch
    m1, m2 = w1.shape[1], w2.shape[1]
    blk = 2000
    b2 = b.reshape(1, f)
    return pl.pallas_call(
        functools.partial(_combo_body, heads, ch),
        grid=(n // blk,),
        in_specs=[pl.BlockSpec((2, blk, f), lambda i: (0, i, 0)),
                  pl.BlockSpec((2, blk, DPAD), lambda i: (0, i, 0)),
                  pl.BlockSpec((1, f), lambda i: (0, 0)),
                  pl.BlockSpec((f, m1), lambda i: (0, 0)),
                  pl.BlockSpec((f, m2), lambda i: (0, 0))],
        out_specs=[pl.BlockSpec((blk, m1), lambda i: (i, 0)),
                   pl.BlockSpec((blk, m2), lambda i: (i, 0))],
        out_shape=[jax.ShapeDtypeStruct((n, m1), jnp.float32),
                   jax.ShapeDtypeStruct((n, m2), jnp.float32)],
    )(op, dp, b2, w1, w2)


def _final_body(heads, ch, op_ref, dp_ref, b_ref, o_ref):
    f = heads * ch
    o = op_ref[0] + op_ref[1]
    d = dp_ref[0] + dp_ref[1]
    db = jnp.dot(d, _head_select(heads, ch, f),
                 preferred_element_type=jnp.float32)
    o_ref[...] = o / (db + 1e-16) + b_ref[...]


def _final(op, dp, b, heads, ch):
    n = op.shape[1]
    f = heads * ch
    blk = 2000
    b2 = b.reshape(1, f)
    return pl.pallas_call(
        functools.partial(_final_body, heads, ch),
        grid=(n // blk,),
        in_specs=[pl.BlockSpec((2, blk, f), lambda i: (0, i, 0)),
                  pl.BlockSpec((2, blk, DPAD), lambda i: (0, i, 0)),
                  pl.BlockSpec((1, f), lambda i: (0, 0))],
        out_specs=pl.BlockSpec((blk, f), lambda i: (i, 0)),
        out_shape=jax.ShapeDtypeStruct((n, f), jnp.float32),
    )(op, dp, b2)


def _edge_sc(xl, xr, src, dst, att, n_dst, heads, ch):
    e = src.shape[0]
    f = heads * ch
    chunk = 64 if f >= 128 else 160
    epw = e // (NC * NS)
    nch = epw // chunk
    ndp = -(-n_dst // (8 * NS)) * (8 * NS)
    rpt = ndp // NS
    rz = 8
    mesh = plsc.VectorSubcoreMesh(core_axis_name="c", subcore_axis_name="s",
                                  num_cores=NC, num_subcores=NS)

    def body(xl_hbm, xr_hbm, src_hbm, dst_hbm, att_hbm, out_hbm, den_hbm,
             sidx0, didx0, xlb0, xrb0, denb0,
             sidx1, didx1, xlb1, xrb1, denb1,
             att_buf, out_sh, den_sh,
             sis0, sid0, sxl0, sxr0, sis1, sid1, sxl1, sxr1):
        c = lax.axis_index("c")
        s = lax.axis_index("s")
        wid = c * NS + s
        pltpu.sync_copy(att_hbm, att_buf)
        sidx = (sidx0, sidx1)
        didx = (didx0, didx1)
        xlb = (xlb0, xlb1)
        xrb = (xrb0, xrb1)
        denb = (denb0, denb1)
        sis = (sis0, sis1)
        sid = (sid0, sid1)
        sxl = (sxl0, sxl1)
        sxr = (sxr0, sxr1)

        def zrow(i, carry):
            for j in range(f // LANES):
                xlb0[i, pl.ds(j * LANES, LANES)] = jnp.zeros(
                    (LANES,), jnp.float32)
            denb0[i, :] = jnp.zeros((LANES,), jnp.float32)
            denb1[i, :] = jnp.zeros((LANES,), jnp.float32)
            return carry
        lax.fori_loop(0, chunk, zrow, 0)
        rbase = pl.multiple_of(s * rpt, 8)

        def zfill(i, carry):
            ro = pl.multiple_of(rbase + i * rz, 8)
            pltpu.sync_copy(xlb0.at[pl.ds(0, rz)], out_sh.at[pl.ds(ro, rz)])
            pltpu.sync_copy(denb0.at[pl.ds(0, rz)], den_sh.at[pl.ds(ro, rz)])
            return carry
        lax.fori_loop(0, rpt // rz, zfill, 0)
        plsc.subcore_barrier()

        lane = lax.iota(jnp.int32, LANES)
        ebase = wid * epw

        def issue_idx(g, b):
            base = pl.multiple_of(ebase + g * chunk, 8)
            pltpu.async_copy(src_hbm.at[pl.ds(base, chunk)], sidx[b], sis[b])
            pltpu.async_copy(dst_hbm.at[pl.ds(base, chunk)], didx[b], sid[b])

        def wait_idx(b):
            pltpu.make_async_copy(src_hbm.at[pl.ds(0, chunk)], sidx[b],
                                  sis[b]).wait()
            pltpu.make_async_copy(dst_hbm.at[pl.ds(0, chunk)], didx[b],
                                  sid[b]).wait()

        def issue_gather(b):
            pltpu.async_copy(xl_hbm.at[sidx[b]], xlb[b], sxl[b])
            pltpu.async_copy(xr_hbm.at[didx[b]], xrb[b], sxr[b])

        def wait_gather(b):
            pltpu.make_async_copy(xl_hbm.at[sidx[b]], xlb[b], sxl[b]).wait()
            pltpu.make_async_copy(xr_hbm.at[didx[b]], xrb[b], sxr[b]).wait()

        U = 8

        def make_group(xl_buf, xr_buf, den_buf):
            def group_body(g, carry):
                rvec = g * LANES + lane
                pe = []
                for h in range(heads):
                    def acc_body(jj, a_c):
                        for k in range(U):
                            col = h * ch + jj * U + k
                            cv = jnp.full((LANES,), col, jnp.int32)
                            a = plsc.load_gather(xl_buf, [rvec, cv])
                            b = plsc.load_gather(xr_buf, [rvec, cv])
                            u = a + b
                            u = jnp.maximum(u, 0.2 * u)
                            a_c = a_c + u * att_buf[col, :]
                        return a_c
                    acc = lax.fori_loop(0, ch // U, acc_body,
                                        jnp.zeros((LANES,), jnp.float32))
                    pe.append(jnp.exp(acc))
                for h in range(heads):
                    hv = jnp.full((LANES,), h, jnp.int32)
                    plsc.store_scatter(den_buf, [rvec, hv], pe[h])
                for h in range(heads):
                    def scale_body(jj, cc):
                        for k in range(U):
                            col = h * ch + jj * U + k
                            cv = jnp.full((LANES,), col, jnp.int32)
                            a = plsc.load_gather(xl_buf, [rvec, cv])
                            plsc.store_scatter(xl_buf, [rvec, cv],
                                               a * pe[h])
                        return cc
                    lax.fori_loop(0, ch // U, scale_body, 0)
                return carry
            return group_body

        groups = (make_group(xlb0, xrb0, denb0), make_group(xlb1, xrb1, denb1))

        issue_idx(0, 0)
        wait_idx(0)
        issue_gather(0)
        issue_idx(1, 1)

        def pair_body(go, carry):
            for b in (0, 1):
                g = 2 * go + b
                nb = 1 - b
                wait_gather(b)

                @pl.when(g + 1 < nch)
                def _():
                    wait_idx(nb)
                    issue_gather(nb)
                lax.fori_loop(0, chunk // LANES, groups[b], 0)
                pltpu.sync_copy(xlb[b], out_sh.at[didx[b]], add=True)
                pltpu.sync_copy(denb[b], den_sh.at[didx[b]], add=True)

                @pl.when(g + 2 < nch)
                def _():
                    issue_idx(g + 2, b)
            return carry

        lax.fori_loop(0, nch // 2, pair_body, 0)
        plsc.subcore_barrier()
        pltpu.sync_copy(out_sh.at[pl.ds(rbase, rpt)],
                        out_hbm.at[c, pl.ds(rbase, rpt)])
        pltpu.sync_copy(den_sh.at[pl.ds(rbase, rpt)],
                        den_hbm.at[c, pl.ds(rbase, rpt)])

    dbuf = [pltpu.VMEM((chunk,), jnp.int32),
            pltpu.VMEM((chunk,), jnp.int32),
            pltpu.VMEM((chunk, f), jnp.float32),
            pltpu.VMEM((chunk, f), jnp.float32),
            pltpu.VMEM((chunk, DPAD), jnp.float32)]
    run = pl.kernel(
        body,
        out_type=[jax.ShapeDtypeStruct((NC, ndp, f), jnp.float32),
                  jax.ShapeDtypeStruct((NC, ndp, DPAD), jnp.float32)],
        mesh=mesh,
        compiler_params=pltpu.CompilerParams(needs_layout_passes=False,
                                             use_tc_tiling_on_sc=False),
        scratch_types=dbuf + dbuf + [
            pltpu.VMEM((f, LANES), jnp.float32),
            pltpu.VMEM_SHARED((ndp, f), jnp.float32),
            pltpu.VMEM_SHARED((ndp, DPAD), jnp.float32),
        ] + [pltpu.SemaphoreType.DMA] * 8,
    )
    att_rep = jnp.tile(att.reshape(-1, 1), (1, LANES))
    op, dp = run(xl, xr, src, dst, att_rep)
    return op[:, :n_dst], dp[:, :n_dst]


def kernel(x_user, x_item, edge_index_user_to_item, edge_index_item_rev_user,
           Wl0_u2i, Wr0_u2i, att0_u2i, b0_u2i,
           Wl0_i2u, Wr0_i2u, att0_i2u, b0_i2u,
           Wl1_u2i, Wr1_u2i, att1_u2i, b1_u2i,
           Wl1_i2u, Wr1_i2u, att1_i2u, b1_i2u):
    n_user = x_user.shape[0]
    n_item = x_item.shape[0]
    heads0, ch0 = att0_u2i.shape
    out1 = att1_u2i.shape[1]

    ne = edge_index_user_to_item.shape[1]
    nep = -(-ne // (NC * NS * 320)) * (NC * NS * 320)
    pad_s = jnp.zeros((nep - ne,), edge_index_user_to_item.dtype)
    pad_d = jnp.full((nep - ne,), n_item, edge_index_user_to_item.dtype)
    src_u2i = jnp.concatenate([edge_index_user_to_item[0], pad_s])
    dst_u2i = jnp.concatenate([edge_index_user_to_item[1], pad_d])
    src_i2u = jnp.concatenate([edge_index_item_rev_user[0], pad_s])
    dst_i2u = jnp.concatenate([edge_index_item_rev_user[1],
                               jnp.full((nep - ne,), n_user,
                                        edge_index_item_rev_user.dtype)])

    xl0_u2i, xr0_i2u = _proj2(x_user, Wl0_u2i, Wr0_i2u)
    xl0_i2u, xr0_u2i = _proj2(x_item, Wl0_i2u, Wr0_u2i)

    op_i0, dp_i0 = _edge_sc(xl0_u2i, xr0_u2i, src_u2i, dst_u2i,
                            att0_u2i.reshape(-1), n_item, heads0, ch0)
    op_u0, dp_u0 = _edge_sc(xl0_i2u, xr0_i2u, src_i2u, dst_i2u,
                            att0_i2u.reshape(-1), n_user, heads0, ch0)

    xl1_u2i, xr1_i2u = _combo(op_u0, dp_u0, b0_i2u, Wl1_u2i, Wr1_i2u,
                              heads0, ch0)
    xl1_i2u, xr1_u2i = _combo(op_i0, dp_i0, b0_u2i, Wl1_i2u, Wr1_u2i,
                              heads0, ch0)

    op_i1, dp_i1 = _edge_sc(xl1_u2i, xr1_u2i, src_u2i, dst_u2i,
                            att1_u2i.reshape(-1), n_item, 1, out1)
    op_u1, dp_u1 = _edge_sc(xl1_i2u, xr1_i2u, src_i2u, dst_i2u,
                            att1_i2u.reshape(-1), n_user, 1, out1)

    out_user = _final(op_u1, dp_u1, b1_i2u, 1, out1)
    out_item = _final(op_i1, dp_i1, b1_u2i, 1, out1)
    return (out_user, out_item)


# speedup vs baseline: 9.8252x; 1.0249x over previous
import functools

import jax
import jax.numpy as jnp
from jax import lax
from jax.experimental import pallas as pl
from jax.experimental.pallas import tpu as pltpu
from jax.experimental.pallas import tpu_sc as plsc

NC = 2
NS = 16
LANES = 16
DPAD = 16


def _proj2_body(x_ref, w1_ref, w2_ref, o1_ref, o2_ref):
    x = x_ref[...]
    o1_ref[...] = jnp.dot(x, w1_ref[...], preferred_element_type=jnp.float32)
    o2_ref[...] = jnp.dot(x, w2_ref[...], preferred_element_type=jnp.float32)


def _proj2(x, w1, w2):
    n, k = x.shape
    blk = 2000
    return pl.pallas_call(
        _proj2_body,
        grid=(n // blk,),
        in_specs=[pl.BlockSpec((blk, k), lambda i: (i, 0)),
                  pl.BlockSpec((k, w1.shape[1]), lambda i: (0, 0)),
                  pl.BlockSpec((k, w2.shape[1]), lambda i: (0, 0))],
        out_specs=[pl.BlockSpec((blk, w1.shape[1]), lambda i: (i, 0)),
                   pl.BlockSpec((blk, w2.shape[1]), lambda i: (i, 0))],
        out_shape=[jax.ShapeDtypeStruct((n, w1.shape[1]), jnp.float32),
                   jax.ShapeDtypeStruct((n, w2.shape[1]), jnp.float32)],
    )(x, w1, w2)


def _head_select(heads, ch, f):
    r = lax.broadcasted_iota(jnp.int32, (DPAD, f), 0)
    c = lax.broadcasted_iota(jnp.int32, (DPAD, f), 1) // ch
    return (r == c).astype(jnp.float32)


def _combo_body(heads, ch, op_ref, dp_ref, b_ref, w1_ref, w2_ref,
                o1_ref, o2_ref):
    f = heads * ch
    o = op_ref[0] + op_ref[1]
    d = dp_ref[0] + dp_ref[1]
    db = jnp.dot(d, _head_select(heads, ch, f),
                 preferred_element_type=jnp.float32)
    h = o / (db + 1e-16) + b_ref[...]
    h = jnp.maximum(h, 0.0)
    o1_ref[...] = jnp.dot(h, w1_ref[...], preferred_element_type=jnp.float32)
    o2_ref[...] = jnp.dot(h, w2_ref[...], preferred_element_type=jnp.float32)


def _combo(op, dp, b, w1, w2, heads, ch):
    n = op.shape[1]
    f = heads * ch
    m1, m2 = w1.shape[1], w2.shape[1]
    blk = 2000
    b2 = b.reshape(1, f)
    return pl.pallas_call(
        functools.partial(_combo_body, heads, ch),
        grid=(n // blk,),
        in_specs=[pl.BlockSpec((2, blk, f), lambda i: (0, i, 0)),
                  pl.BlockSpec((2, blk, DPAD), lambda i: (0, i, 0)),
                  pl.BlockSpec((1, f), lambda i: (0, 0)),
                  pl.BlockSpec((f, m1), lambda i: (0, 0)),
                  pl.BlockSpec((f, m2), lambda i: (0, 0))],
        out_specs=[pl.BlockSpec((blk, m1), lambda i: (i, 0)),
                   pl.BlockSpec((blk, m2), lambda i: (i, 0))],
        out_shape=[jax.ShapeDtypeStruct((n, m1), jnp.float32),
                   jax.ShapeDtypeStruct((n, m2), jnp.float32)],
    )(op, dp, b2, w1, w2)


def _final_body(heads, ch, op_ref, dp_ref, b_ref, o_ref):
    f = heads * ch
    o = op_ref[0] + op_ref[1]
    d = dp_ref[0] + dp_ref[1]
    db = jnp.dot(d, _head_select(heads, ch, f),
                 preferred_element_type=jnp.float32)
    o_ref[...] = o / (db + 1e-16) + b_ref[...]


def _final(op, dp, b, heads, ch):
    n = op.shape[1]
    f = heads * ch
    blk = 2000
    b2 = b.reshape(1, f)
    return pl.pallas_call(
        functools.partial(_final_body, heads, ch),
        grid=(n // blk,),
        in_specs=[pl.BlockSpec((2, blk, f), lambda i: (0, i, 0)),
                  pl.BlockSpec((2, blk, DPAD), lambda i: (0, i, 0)),
                  pl.BlockSpec((1, f), lambda i: (0, 0))],
        out_specs=pl.BlockSpec((blk, f), lambda i: (i, 0)),
        out_shape=jax.ShapeDtypeStruct((n, f), jnp.float32),
    )(op, dp, b2)


def _edge_sc(xl, xr, src, dst, att, n_dst, heads, ch):
    e = src.shape[0]
    f = heads * ch
    chunk = 64 if f >= 128 else 160
    epw = e // (NC * NS)
    nch = epw // chunk
    ndp = -(-n_dst // (8 * NS)) * (8 * NS)
    rpt = ndp // NS
    rz = 8
    mesh = plsc.VectorSubcoreMesh(core_axis_name="c", subcore_axis_name="s",
                                  num_cores=NC, num_subcores=NS)

    def body(xl_hbm, xr_hbm, ei_hbm, att_hbm, out_hbm, den_hbm,
             eib0, xlb0, xrb0, denb0, sdx0,
             eib1, xlb1, xrb1, denb1, sdx1,
             att_buf, out_sh, den_sh,
             six0, sxl0, sxr0, ssl0, ssd0,
             six1, sxl1, sxr1, ssl1, ssd1):
        c = lax.axis_index("c")
        s = lax.axis_index("s")
        wid = c * NS + s
        pltpu.sync_copy(att_hbm, att_buf)
        eib = (eib0, eib1)
        sdx = (sdx0, sdx1)
        xlb = (xlb0, xlb1)
        xrb = (xrb0, xrb1)
        denb = (denb0, denb1)
        six = (six0, six1)
        sxl = (sxl0, sxl1)
        sxr = (sxr0, sxr1)
        ssl = (ssl0, ssl1)
        ssd = (ssd0, ssd1)

        def zrow(i, carry):
            for j in range(f // LANES):
                xlb0[i, pl.ds(j * LANES, LANES)] = jnp.zeros(
                    (LANES,), jnp.float32)
            denb0[i, :] = jnp.zeros((LANES,), jnp.float32)
            denb1[i, :] = jnp.zeros((LANES,), jnp.float32)
            return carry
        lax.fori_loop(0, chunk, zrow, 0)
        rbase = pl.multiple_of(s * rpt, 8)

        def zfill(i, carry):
            ro = pl.multiple_of(rbase + i * rz, 8)
            pltpu.sync_copy(xlb0.at[pl.ds(0, rz)], out_sh.at[pl.ds(ro, rz)])
            pltpu.sync_copy(denb0.at[pl.ds(0, rz)], den_sh.at[pl.ds(ro, rz)])
            return carry
        lax.fori_loop(0, rpt // rz, zfill, 0)
        plsc.subcore_barrier()

        lane = lax.iota(jnp.int32, LANES)
        ebase = wid * epw

        def issue_idx(g, b):
            base = pl.multiple_of(ebase + g * chunk, 8)
            pltpu.async_copy(ei_hbm.at[:, pl.ds(base, chunk)], eib[b], six[b])

        def wait_idx(b):
            pltpu.make_async_copy(ei_hbm.at[:, pl.ds(0, chunk)], eib[b],
                                  six[b]).wait()

        def issue_gather(b):
            pltpu.async_copy(xl_hbm.at[eib[b].at[0]], xlb[b], sxl[b])
            pltpu.async_copy(xr_hbm.at[eib[b].at[1]], xrb[b], sxr[b])

        def wait_gather(b):
            pltpu.make_async_copy(xl_hbm.at[eib[b].at[0]], xlb[b],
                                  sxl[b]).wait()
            pltpu.make_async_copy(xr_hbm.at[eib[b].at[1]], xrb[b],
                                  sxr[b]).wait()

        def issue_scatter(b):
            pltpu.async_copy(xlb[b], out_sh.at[sdx[b]], ssl[b], add=True)
            pltpu.async_copy(denb[b], den_sh.at[sdx[b]], ssd[b], add=True)

        def wait_scatter(b):
            pltpu.make_async_copy(xlb[b], out_sh.at[sdx[b]], ssl[b]).wait()
            pltpu.make_async_copy(denb[b], den_sh.at[sdx[b]], ssd[b]).wait()

        U = 8

        def make_group(xl_buf, xr_buf, den_buf):
            def group_body(g, carry):
                rvec = g * LANES + lane
                pe = []
                for h in range(heads):
                    def acc_body(jj, a_c):
                        for k in range(U):
                            col = h * ch + jj * U + k
                            cv = jnp.full((LANES,), col, jnp.int32)
                            a = plsc.load_gather(xl_buf, [rvec, cv])
                            b = plsc.load_gather(xr_buf, [rvec, cv])
                            u = a + b
                            u = jnp.maximum(u, 0.2 * u)
                            a_c = a_c + u * att_buf[col, :]
                        return a_c
                    acc = lax.fori_loop(0, ch // U, acc_body,
                                        jnp.zeros((LANES,), jnp.float32))
                    pe.append(jnp.exp(acc))
                for h in range(heads):
                    hv = jnp.full((LANES,), h, jnp.int32)
                    plsc.store_scatter(den_buf, [rvec, hv], pe[h])
                for h in range(heads):
                    def scale_body(jj, cc):
                        for k in range(U):
                            col = h * ch + jj * U + k
                            cv = jnp.full((LANES,), col, jnp.int32)
                            a = plsc.load_gather(xl_buf, [rvec, cv])
                            plsc.store_scatter(xl_buf, [rvec, cv],
                                               a * pe[h])
                        return cc
                    lax.fori_loop(0, ch // U, scale_body, 0)
                return carry
            return group_body

        groups = (make_group(xlb0, xrb0, denb0), make_group(xlb1, xrb1, denb1))

        issue_idx(0, 0)
        wait_idx(0)
        issue_gather(0)
        issue_idx(1, 1)

        def pair_body(go, carry):
            for b in (0, 1):
                g = 2 * go + b
                nb = 1 - b
                wait_gather(b)

                @pl.when((g + 1 < nch) & (g >= 1))
                def _():
                    wait_scatter(nb)

                @pl.when(g + 1 < nch)
                def _():
                    wait_idx(nb)
                    issue_gather(nb)
                for i in range(chunk // LANES):
                    sdx[b][pl.ds(i * LANES, LANES)] = (
                        eib[b][1, pl.ds(i * LANES, LANES)])

                @pl.when(g + 2 < nch)
                def _():
                    issue_idx(g + 2, b)
                lax.fori_loop(0, chunk // LANES, groups[b], 0)
                issue_scatter(b)
            return carry

        lax.fori_loop(0, nch // 2, pair_body, 0)
        wait_scatter(0)
        wait_scatter(1)
        plsc.subcore_barrier()
        pltpu.sync_copy(out_sh.at[pl.ds(rbase, rpt)],
                        out_hbm.at[c, pl.ds(rbase, rpt)])
        pltpu.sync_copy(den_sh.at[pl.ds(rbase, rpt)],
                        den_hbm.at[c, pl.ds(rbase, rpt)])

    dbuf = [pltpu.VMEM((2, chunk), jnp.int32),
            pltpu.VMEM((chunk, f), jnp.float32),
            pltpu.VMEM((chunk, f), jnp.float32),
            pltpu.VMEM((chunk, DPAD), jnp.float32),
            pltpu.VMEM((chunk,), jnp.int32)]
    run = pl.kernel(
        body,
        out_type=[jax.ShapeDtypeStruct((NC, ndp, f), jnp.float32),
                  jax.ShapeDtypeStruct((NC, ndp, DPAD), jnp.float32)],
        mesh=mesh,
        compiler_params=pltpu.CompilerParams(needs_layout_passes=False,
                                             use_tc_tiling_on_sc=False),
        scratch_types=dbuf + dbuf + [
            pltpu.VMEM((f, LANES), jnp.float32),
            pltpu.VMEM_SHARED((ndp, f), jnp.float32),
            pltpu.VMEM_SHARED((ndp, DPAD), jnp.float32),
        ] + [pltpu.SemaphoreType.DMA] * 10,
    )
    att_rep = jnp.tile(att.reshape(-1, 1), (1, LANES))
    ei = jnp.stack([src, dst])
    op, dp = run(xl, xr, ei, att_rep)
    return op[:, :n_dst], dp[:, :n_dst]


def kernel(x_user, x_item, edge_index_user_to_item, edge_index_item_rev_user,
           Wl0_u2i, Wr0_u2i, att0_u2i, b0_u2i,
           Wl0_i2u, Wr0_i2u, att0_i2u, b0_i2u,
           Wl1_u2i, Wr1_u2i, att1_u2i, b1_u2i,
           Wl1_i2u, Wr1_i2u, att1_i2u, b1_i2u):
    n_user = x_user.shape[0]
    n_item = x_item.shape[0]
    heads0, ch0 = att0_u2i.shape
    out1 = att1_u2i.shape[1]

    ne = edge_index_user_to_item.shape[1]
    nep = -(-ne // (NC * NS * 320)) * (NC * NS * 320)
    pad_s = jnp.zeros((nep - ne,), edge_index_user_to_item.dtype)
    pad_d = jnp.full((nep - ne,), n_item, edge_index_user_to_item.dtype)
    src_u2i = jnp.concatenate([edge_index_user_to_item[0], pad_s])
    dst_u2i = jnp.concatenate([edge_index_user_to_item[1], pad_d])
    src_i2u = jnp.concatenate([edge_index_item_rev_user[0], pad_s])
    dst_i2u = jnp.concatenate([edge_index_item_rev_user[1],
                               jnp.full((nep - ne,), n_user,
                                        edge_index_item_rev_user.dtype)])

    xl0_u2i, xr0_i2u = _proj2(x_user, Wl0_u2i, Wr0_i2u)
    xl0_i2u, xr0_u2i = _proj2(x_item, Wl0_i2u, Wr0_u2i)

    op_i0, dp_i0 = _edge_sc(xl0_u2i, xr0_u2i, src_u2i, dst_u2i,
                            att0_u2i.reshape(-1), n_item, heads0, ch0)
    op_u0, dp_u0 = _edge_sc(xl0_i2u, xr0_i2u, src_i2u, dst_i2u,
                            att0_i2u.reshape(-1), n_user, heads0, ch0)

    xl1_u2i, xr1_i2u = _combo(op_u0, dp_u0, b0_i2u, Wl1_u2i, Wr1_i2u,
                              heads0, ch0)
    xl1_i2u, xr1_u2i = _combo(op_i0, dp_i0, b0_u2i, Wl1_i2u, Wr1_u2i,
                              heads0, ch0)

    op_i1, dp_i1 = _edge_sc(xl1_u2i, xr1_u2i, src_u2i, dst_u2i,
                            att1_u2i.reshape(-1), n_item, 1, out1)
    op_u1, dp_u1 = _edge_sc(xl1_i2u, xr1_i2u, src_i2u, dst_i2u,
                            att1_i2u.reshape(-1), n_user, 1, out1)

    out_user = _final(op_u1, dp_u1, b1_i2u, 1, out1)
    out_item = _final(op_i1, dp_i1, b1_u2i, 1, out1)
    return (out_user, out_item)


# speedup vs baseline: 39.1151x; 3.9811x over previous
import functools

import jax
import jax.numpy as jnp
from jax import lax
from jax.experimental import pallas as pl
from jax.experimental.pallas import tpu as pltpu
from jax.experimental.pallas import tpu_sc as plsc

NC = 2
NS = 16
LANES = 16
DPAD = 16


def _proj2_body(x_ref, w1_ref, w2_ref, o1_ref, o2_ref):
    x = x_ref[...]
    o1_ref[...] = jnp.dot(x, w1_ref[...], preferred_element_type=jnp.float32)
    o2_ref[...] = jnp.dot(x, w2_ref[...], preferred_element_type=jnp.float32)


def _proj2(x, w1, w2):
    n, k = x.shape
    blk = 2000
    return pl.pallas_call(
        _proj2_body,
        grid=(n // blk,),
        in_specs=[pl.BlockSpec((blk, k), lambda i: (i, 0)),
                  pl.BlockSpec((k, w1.shape[1]), lambda i: (0, 0)),
                  pl.BlockSpec((k, w2.shape[1]), lambda i: (0, 0))],
        out_specs=[pl.BlockSpec((blk, w1.shape[1]), lambda i: (i, 0)),
                   pl.BlockSpec((blk, w2.shape[1]), lambda i: (i, 0))],
        out_shape=[jax.ShapeDtypeStruct((n, w1.shape[1]), jnp.float32),
                   jax.ShapeDtypeStruct((n, w2.shape[1]), jnp.float32)],
    )(x, w1, w2)


def _head_select(heads, ch, f):
    r = lax.broadcasted_iota(jnp.int32, (DPAD, f), 0)
    c = lax.broadcasted_iota(jnp.int32, (DPAD, f), 1) // ch
    return (r == c).astype(jnp.float32)


def _combo_body(heads, ch, op_ref, dp_ref, b_ref, w1_ref, w2_ref,
                o1_ref, o2_ref):
    f = heads * ch
    o = op_ref[0] + op_ref[1]
    d = dp_ref[0] + dp_ref[1]
    db = jnp.dot(d, _head_select(heads, ch, f),
                 preferred_element_type=jnp.float32)
    h = o / (db + 1e-16) + b_ref[...]
    h = jnp.maximum(h, 0.0)
    o1_ref[...] = jnp.dot(h, w1_ref[...], preferred_element_type=jnp.float32)
    o2_ref[...] = jnp.dot(h, w2_ref[...], preferred_element_type=jnp.float32)


def _combo(op, dp, b, w1, w2, heads, ch):
    n = op.shape[1]
    f = heads * ch
    m1, m2 = w1.shape[1], w2.shape[1]
    blk = 2000
    b2 = b.reshape(1, f)
    return pl.pallas_call(
        functools.partial(_combo_body, heads, ch),
        grid=(n // blk,),
        in_specs=[pl.BlockSpec((2, blk, f), lambda i: (0, i, 0)),
                  pl.BlockSpec((2, blk, DPAD), lambda i: (0, i, 0)),
                  pl.BlockSpec((1, f), lambda i: (0, 0)),
                  pl.BlockSpec((f, m1), lambda i: (0, 0)),
                  pl.BlockSpec((f, m2), lambda i: (0, 0))],
        out_specs=[pl.BlockSpec((blk, m1), lambda i: (i, 0)),
                   pl.BlockSpec((blk, m2), lambda i: (i, 0))],
        out_shape=[jax.ShapeDtypeStruct((n, m1), jnp.float32),
                   jax.ShapeDtypeStruct((n, m2), jnp.float32)],
    )(op, dp, b2, w1, w2)


def _final_body(heads, ch, op_ref, dp_ref, b_ref, o_ref):
    f = heads * ch
    o = op_ref[0] + op_ref[1]
    d = dp_ref[0] + dp_ref[1]
    db = jnp.dot(d, _head_select(heads, ch, f),
                 preferred_element_type=jnp.float32)
    o_ref[...] = o / (db + 1e-16) + b_ref[...]


def _final(op, dp, b, heads, ch):
    n = op.shape[1]
    f = heads * ch
    blk = 2000
    b2 = b.reshape(1, f)
    return pl.pallas_call(
        functools.partial(_final_body, heads, ch),
        grid=(n // blk,),
        in_specs=[pl.BlockSpec((2, blk, f), lambda i: (0, i, 0)),
                  pl.BlockSpec((2, blk, DPAD), lambda i: (0, i, 0)),
                  pl.BlockSpec((1, f), lambda i: (0, 0))],
        out_specs=pl.BlockSpec((blk, f), lambda i: (i, 0)),
        out_shape=jax.ShapeDtypeStruct((n, f), jnp.float32),
    )(op, dp, b2)


def _edge_sc(xl, xr, src, dst, att, n_dst, heads, ch):
    e = src.shape[0]
    f = heads * ch
    chunk = 64 if f >= 128 else 160
    epw = e // (NC * NS)
    nch = epw // chunk
    ndp = -(-n_dst // (8 * NS)) * (8 * NS)
    rpt = ndp // NS
    rz = 8
    mesh = plsc.VectorSubcoreMesh(core_axis_name="c", subcore_axis_name="s",
                                  num_cores=NC, num_subcores=NS)

    def body(xl_hbm, xr_hbm, ei_hbm, att_hbm, out_hbm, den_hbm,
             eib0, xlb0, xrb0, denb0, sdx0,
             eib1, xlb1, xrb1, denb1, sdx1,
             att_buf, out_sh, den_sh,
             six0, sxl0, sxr0, ssl0, ssd0,
             six1, sxl1, sxr1, ssl1, ssd1):
        c = lax.axis_index("c")
        s = lax.axis_index("s")
        wid = c * NS + s
        pltpu.sync_copy(att_hbm, att_buf)
        eib = (eib0, eib1)
        sdx = (sdx0, sdx1)
        xlb = (xlb0, xlb1)
        xrb = (xrb0, xrb1)
        denb = (denb0, denb1)
        six = (six0, six1)
        sxl = (sxl0, sxl1)
        sxr = (sxr0, sxr1)
        ssl = (ssl0, ssl1)
        ssd = (ssd0, ssd1)

        def zrow(i, carry):
            for j in range(f // LANES):
                xlb0[i, pl.ds(j * LANES, LANES)] = jnp.zeros(
                    (LANES,), jnp.float32)
            denb0[i, :] = jnp.zeros((LANES,), jnp.float32)
            denb1[i, :] = jnp.zeros((LANES,), jnp.float32)
            return carry
        lax.fori_loop(0, chunk, zrow, 0)
        rbase = pl.multiple_of(s * rpt, 8)

        def zfill(i, carry):
            ro = pl.multiple_of(rbase + i * rz, 8)
            pltpu.sync_copy(xlb0.at[pl.ds(0, rz)], out_sh.at[pl.ds(ro, rz)])
            pltpu.sync_copy(denb0.at[pl.ds(0, rz)], den_sh.at[pl.ds(ro, rz)])
            return carry
        lax.fori_loop(0, rpt // rz, zfill, 0)
        plsc.subcore_barrier()

        lane = lax.iota(jnp.int32, LANES)
        ebase = wid * epw

        def issue_idx(g, b):
            base = pl.multiple_of(ebase + g * chunk, 8)
            pltpu.async_copy(ei_hbm.at[:, pl.ds(base, chunk)], eib[b], six[b])

        def wait_idx(b):
            pltpu.make_async_copy(ei_hbm.at[:, pl.ds(0, chunk)], eib[b],
                                  six[b]).wait()

        def issue_gather(b):
            pltpu.async_copy(xl_hbm.at[eib[b].at[0]], xlb[b], sxl[b])
            pltpu.async_copy(xr_hbm.at[eib[b].at[1]], xrb[b], sxr[b])

        def wait_gather(b):
            pltpu.make_async_copy(xl_hbm.at[eib[b].at[0]], xlb[b],
                                  sxl[b]).wait()
            pltpu.make_async_copy(xr_hbm.at[eib[b].at[1]], xrb[b],
                                  sxr[b]).wait()

        def issue_scatter(b):
            pltpu.async_copy(xlb[b], out_sh.at[sdx[b]], ssl[b], add=True)
            pltpu.async_copy(denb[b], den_sh.at[sdx[b]], ssd[b], add=True)

        def wait_scatter(b):
            pltpu.make_async_copy(xlb[b], out_sh.at[sdx[b]], ssl[b]).wait()
            pltpu.make_async_copy(denb[b], den_sh.at[sdx[b]], ssd[b]).wait()

        U = 8
        TPB = LANES // U

        def make_group(xl_buf, xr_buf, den_buf):
            def group_body(g, carry):
                rvec = g * LANES + lane
                pe = []
                for h in range(heads):
                    def acc_body(jj, a_c, h=h):
                        b16 = jj // TPB
                        kk = jj - b16 * TPB
                        cb = h * ch + b16 * LANES
                        for k2 in range(U):
                            k = kk * U + k2
                            cv = cb + ((lane + k) & (LANES - 1))
                            a = plsc.load_gather(xl_buf, [rvec, cv])
                            b = plsc.load_gather(xr_buf, [rvec, cv])
                            u = a + b
                            u = jnp.maximum(u, 0.2 * u)
                            a_c = a_c + u * att_buf[cb + k, :]
                        return a_c
                    acc = lax.fori_loop(0, (ch // LANES) * TPB, acc_body,
                                        jnp.zeros((LANES,), jnp.float32))
                    pe.append(jnp.exp(acc))
                for h in range(heads):
                    hv = jnp.full((LANES,), h, jnp.int32)
                    plsc.store_scatter(den_buf, [rvec, hv], pe[h])
                for h in range(heads):
                    def scale_body(jj, cc, h=h):
                        b16 = jj // TPB
                        kk = jj - b16 * TPB
                        cb = h * ch + b16 * LANES
                        for k2 in range(U):
                            k = kk * U + k2
                            cv = cb + ((lane + k) & (LANES - 1))
                            a = plsc.load_gather(xl_buf, [rvec, cv])
                            plsc.store_scatter(xl_buf, [rvec, cv],
                                               a * pe[h])
                        return cc
                    lax.fori_loop(0, (ch // LANES) * TPB, scale_body, 0)
                return carry
            return group_body

        groups = (make_group(xlb0, xrb0, denb0), make_group(xlb1, xrb1, denb1))

        issue_idx(0, 0)
        wait_idx(0)
        issue_gather(0)
        issue_idx(1, 1)

        def pair_body(go, carry):
            for b in (0, 1):
                g = 2 * go + b
                nb = 1 - b
                wait_gather(b)

                @pl.when((g + 1 < nch) & (g >= 1))
                def _():
                    wait_scatter(nb)

                @pl.when(g + 1 < nch)
                def _():
                    wait_idx(nb)
                    issue_gather(nb)
                for i in range(chunk // LANES):
                    sdx[b][pl.ds(i * LANES, LANES)] = (
                        eib[b][1, pl.ds(i * LANES, LANES)])

                @pl.when(g + 2 < nch)
                def _():
                    issue_idx(g + 2, b)
                lax.fori_loop(0, chunk // LANES, groups[b], 0)
                issue_scatter(b)
            return carry

        lax.fori_loop(0, nch // 2, pair_body, 0)
        wait_scatter(0)
        wait_scatter(1)
        plsc.subcore_barrier()
        pltpu.sync_copy(out_sh.at[pl.ds(rbase, rpt)],
                        out_hbm.at[c, pl.ds(rbase, rpt)])
        pltpu.sync_copy(den_sh.at[pl.ds(rbase, rpt)],
                        den_hbm.at[c, pl.ds(rbase, rpt)])

    dbuf = [pltpu.VMEM((2, chunk), jnp.int32),
            pltpu.VMEM((chunk, f), jnp.float32),
            pltpu.VMEM((chunk, f), jnp.float32),
            pltpu.VMEM((chunk, DPAD), jnp.float32),
            pltpu.VMEM((chunk,), jnp.int32)]
    run = pl.kernel(
        body,
        out_type=[jax.ShapeDtypeStruct((NC, ndp, f), jnp.float32),
                  jax.ShapeDtypeStruct((NC, ndp, DPAD), jnp.float32)],
        mesh=mesh,
        compiler_params=pltpu.CompilerParams(needs_layout_passes=False,
                                             use_tc_tiling_on_sc=False),
        scratch_types=dbuf + dbuf + [
            pltpu.VMEM((f, LANES), jnp.float32),
            pltpu.VMEM_SHARED((ndp, f), jnp.float32),
            pltpu.VMEM_SHARED((ndp, DPAD), jnp.float32),
        ] + [pltpu.SemaphoreType.DMA] * 10,
    )
    rows = jnp.arange(f)
    cols = (rows // LANES * LANES)[:, None] + (
        (rows % LANES)[:, None] + jnp.arange(LANES)[None, :]) % LANES
    att_rot = att.reshape(-1)[cols]
    ei = jnp.stack([src, dst])
    op, dp = run(xl, xr, ei, att_rot)
    return op[:, :n_dst], dp[:, :n_dst]


def kernel(x_user, x_item, edge_index_user_to_item, edge_index_item_rev_user,
           Wl0_u2i, Wr0_u2i, att0_u2i, b0_u2i,
           Wl0_i2u, Wr0_i2u, att0_i2u, b0_i2u,
           Wl1_u2i, Wr1_u2i, att1_u2i, b1_u2i,
           Wl1_i2u, Wr1_i2u, att1_i2u, b1_i2u):
    n_user = x_user.shape[0]
    n_item = x_item.shape[0]
    heads0, ch0 = att0_u2i.shape
    out1 = att1_u2i.shape[1]

    ne = edge_index_user_to_item.shape[1]
    nep = -(-ne // (NC * NS * 320)) * (NC * NS * 320)
    pad_s = jnp.zeros((nep - ne,), edge_index_user_to_item.dtype)
    pad_d = jnp.full((nep - ne,), n_item, edge_index_user_to_item.dtype)
    src_u2i = jnp.concatenate([edge_index_user_to_item[0], pad_s])
    dst_u2i = jnp.concatenate([edge_index_user_to_item[1], pad_d])
    src_i2u = jnp.concatenate([edge_index_item_rev_user[0], pad_s])
    dst_i2u = jnp.concatenate([edge_index_item_rev_user[1],
                               jnp.full((nep - ne,), n_user,
                                        edge_index_item_rev_user.dtype)])

    xl0_u2i, xr0_i2u = _proj2(x_user, Wl0_u2i, Wr0_i2u)
    xl0_i2u, xr0_u2i = _proj2(x_item, Wl0_i2u, Wr0_u2i)

    op_i0, dp_i0 = _edge_sc(xl0_u2i, xr0_u2i, src_u2i, dst_u2i,
                            att0_u2i.reshape(-1), n_item, heads0, ch0)
    op_u0, dp_u0 = _edge_sc(xl0_i2u, xr0_i2u, src_i2u, dst_i2u,
                            att0_i2u.reshape(-1), n_user, heads0, ch0)

    xl1_u2i, xr1_i2u = _combo(op_u0, dp_u0, b0_i2u, Wl1_u2i, Wr1_i2u,
                              heads0, ch0)
    xl1_i2u, xr1_u2i = _combo(op_i0, dp_i0, b0_u2i, Wl1_i2u, Wr1_u2i,
                              heads0, ch0)

    op_i1, dp_i1 = _edge_sc(xl1_u2i, xr1_u2i, src_u2i, dst_u2i,
                            att1_u2i.reshape(-1), n_item, 1, out1)
    op_u1, dp_u1 = _edge_sc(xl1_i2u, xr1_i2u, src_i2u, dst_i2u,
                            att1_i2u.reshape(-1), n_user, 1, out1)

    out_user = _final(op_u1, dp_u1, b1_i2u, 1, out1)
    out_item = _final(op_i1, dp_i1, b1_u2i, 1, out1)
    return (out_user, out_item)


# speedup vs baseline: 39.2123x; 1.0025x over previous
import functools

import jax
import jax.numpy as jnp
from jax import lax
from jax.experimental import pallas as pl
from jax.experimental.pallas import tpu as pltpu
from jax.experimental.pallas import tpu_sc as plsc

NC = 2
NS = 16
LANES = 16
DPAD = 16


def _proj2_body(x_ref, w1_ref, w2_ref, o1_ref, o2_ref):
    x = x_ref[...]
    o1_ref[...] = jnp.dot(x, w1_ref[...], preferred_element_type=jnp.float32)
    o2_ref[...] = jnp.dot(x, w2_ref[...], preferred_element_type=jnp.float32)


def _proj2(x, w1, w2):
    n, k = x.shape
    blk = 2000
    return pl.pallas_call(
        _proj2_body,
        grid=(n // blk,),
        in_specs=[pl.BlockSpec((blk, k), lambda i: (i, 0)),
                  pl.BlockSpec((k, w1.shape[1]), lambda i: (0, 0)),
                  pl.BlockSpec((k, w2.shape[1]), lambda i: (0, 0))],
        out_specs=[pl.BlockSpec((blk, w1.shape[1]), lambda i: (i, 0)),
                   pl.BlockSpec((blk, w2.shape[1]), lambda i: (i, 0))],
        out_shape=[jax.ShapeDtypeStruct((n, w1.shape[1]), jnp.float32),
                   jax.ShapeDtypeStruct((n, w2.shape[1]), jnp.float32)],
    )(x, w1, w2)


def _head_select(heads, ch, f):
    r = lax.broadcasted_iota(jnp.int32, (DPAD, f), 0)
    c = lax.broadcasted_iota(jnp.int32, (DPAD, f), 1) // ch
    return (r == c).astype(jnp.float32)


def _combo_body(heads, ch, op_ref, dp_ref, b_ref, w1_ref, w2_ref,
                o1_ref, o2_ref):
    f = heads * ch
    o = op_ref[0] + op_ref[1]
    d = dp_ref[0] + dp_ref[1]
    db = jnp.dot(d, _head_select(heads, ch, f),
                 preferred_element_type=jnp.float32)
    h = o / (db + 1e-16) + b_ref[...]
    h = jnp.maximum(h, 0.0)
    o1_ref[...] = jnp.dot(h, w1_ref[...], preferred_element_type=jnp.float32)
    o2_ref[...] = jnp.dot(h, w2_ref[...], preferred_element_type=jnp.float32)


def _combo(op, dp, b, w1, w2, heads, ch):
    n = op.shape[1]
    f = heads * ch
    m1, m2 = w1.shape[1], w2.shape[1]
    blk = 2000
    b2 = b.reshape(1, f)
    return pl.pallas_call(
        functools.partial(_combo_body, heads, ch),
        grid=(n // blk,),
        in_specs=[pl.BlockSpec((2, blk, f), lambda i: (0, i, 0)),
                  pl.BlockSpec((2, blk, DPAD), lambda i: (0, i, 0)),
                  pl.BlockSpec((1, f), lambda i: (0, 0)),
                  pl.BlockSpec((f, m1), lambda i: (0, 0)),
                  pl.BlockSpec((f, m2), lambda i: (0, 0))],
        out_specs=[pl.BlockSpec((blk, m1), lambda i: (i, 0)),
                   pl.BlockSpec((blk, m2), lambda i: (i, 0))],
        out_shape=[jax.ShapeDtypeStruct((n, m1), jnp.float32),
                   jax.ShapeDtypeStruct((n, m2), jnp.float32)],
    )(op, dp, b2, w1, w2)


def _final_body(heads, ch, op_ref, dp_ref, b_ref, o_ref):
    f = heads * ch
    o = op_ref[0] + op_ref[1]
    d = dp_ref[0] + dp_ref[1]
    db = jnp.dot(d, _head_select(heads, ch, f),
                 preferred_element_type=jnp.float32)
    o_ref[...] = o / (db + 1e-16) + b_ref[...]


def _final(op, dp, b, heads, ch):
    n = op.shape[1]
    f = heads * ch
    blk = 2000
    b2 = b.reshape(1, f)
    return pl.pallas_call(
        functools.partial(_final_body, heads, ch),
        grid=(n // blk,),
        in_specs=[pl.BlockSpec((2, blk, f), lambda i: (0, i, 0)),
                  pl.BlockSpec((2, blk, DPAD), lambda i: (0, i, 0)),
                  pl.BlockSpec((1, f), lambda i: (0, 0))],
        out_specs=pl.BlockSpec((blk, f), lambda i: (i, 0)),
        out_shape=jax.ShapeDtypeStruct((n, f), jnp.float32),
    )(op, dp, b2)


def _edge_sc(xl, xr, src, dst, att, n_dst, heads, ch):
    e = src.shape[0]
    f = heads * ch
    chunk = 64 if f >= 128 else 160
    epw = e // (NC * NS)
    nch = epw // chunk
    ndp = -(-n_dst // (8 * NS)) * (8 * NS)
    rpt = ndp // NS
    rz = 8
    mesh = plsc.VectorSubcoreMesh(core_axis_name="c", subcore_axis_name="s",
                                  num_cores=NC, num_subcores=NS)

    def body(xl_hbm, xr_hbm, ei_hbm, att_hbm, out_hbm, den_hbm,
             eib0, xlb0, xrb0, denb0, sdx0,
             eib1, xlb1, xrb1, denb1, sdx1,
             att_buf, out_sh, den_sh,
             six0, sxl0, sxr0, ssl0, ssd0,
             six1, sxl1, sxr1, ssl1, ssd1):
        c = lax.axis_index("c")
        s = lax.axis_index("s")
        wid = c * NS + s
        pltpu.sync_copy(att_hbm, att_buf)
        eib = (eib0, eib1)
        sdx = (sdx0, sdx1)
        xlb = (xlb0, xlb1)
        xrb = (xrb0, xrb1)
        denb = (denb0, denb1)
        six = (six0, six1)
        sxl = (sxl0, sxl1)
        sxr = (sxr0, sxr1)
        ssl = (ssl0, ssl1)
        ssd = (ssd0, ssd1)

        def zrow(i, carry):
            for j in range(f // LANES):
                xlb0[i, pl.ds(j * LANES, LANES)] = jnp.zeros(
                    (LANES,), jnp.float32)
            denb0[i, :] = jnp.zeros((LANES,), jnp.float32)
            denb1[i, :] = jnp.zeros((LANES,), jnp.float32)
            return carry
        lax.fori_loop(0, chunk, zrow, 0)
        rbase = pl.multiple_of(s * rpt, 8)

        def zfill(i, carry):
            ro = pl.multiple_of(rbase + i * rz, 8)
            pltpu.sync_copy(xlb0.at[pl.ds(0, rz)], out_sh.at[pl.ds(ro, rz)])
            pltpu.sync_copy(denb0.at[pl.ds(0, rz)], den_sh.at[pl.ds(ro, rz)])
            return carry
        lax.fori_loop(0, rpt // rz, zfill, 0)
        plsc.subcore_barrier()

        lane = lax.iota(jnp.int32, LANES)
        ebase = wid * epw

        def issue_idx(g, b):
            base = pl.multiple_of(ebase + g * chunk, 8)
            pltpu.async_copy(ei_hbm.at[:, pl.ds(base, chunk)], eib[b], six[b])

        def wait_idx(b):
            pltpu.make_async_copy(ei_hbm.at[:, pl.ds(0, chunk)], eib[b],
                                  six[b]).wait()

        def issue_gather(b):
            pltpu.async_copy(xl_hbm.at[eib[b].at[0]], xlb[b], sxl[b])
            pltpu.async_copy(xr_hbm.at[eib[b].at[1]], xrb[b], sxr[b])

        def wait_gather(b):
            pltpu.make_async_copy(xl_hbm.at[eib[b].at[0]], xlb[b],
                                  sxl[b]).wait()
            pltpu.make_async_copy(xr_hbm.at[eib[b].at[1]], xrb[b],
                                  sxr[b]).wait()

        def issue_scatter(b):
            pltpu.async_copy(xlb[b], out_sh.at[sdx[b]], ssl[b], add=True)
            pltpu.async_copy(denb[b], den_sh.at[sdx[b]], ssd[b], add=True)

        def wait_scatter(b):
            pltpu.make_async_copy(xlb[b], out_sh.at[sdx[b]], ssl[b]).wait()
            pltpu.make_async_copy(denb[b], den_sh.at[sdx[b]], ssd[b]).wait()

        U = 16
        TPB = LANES // U

        def make_group(xl_buf, xr_buf, den_buf):
            def group_body(g, carry):
                rvec = g * LANES + lane
                pe = []
                for h in range(heads):
                    def acc_body(jj, a_c, h=h):
                        b16 = jj // TPB
                        kk = jj - b16 * TPB
                        cb = h * ch + b16 * LANES
                        for k2 in range(U):
                            k = kk * U + k2
                            cv = cb + ((lane + k) & (LANES - 1))
                            a = plsc.load_gather(xl_buf, [rvec, cv])
                            b = plsc.load_gather(xr_buf, [rvec, cv])
                            u = a + b
                            u = jnp.maximum(u, 0.2 * u)
                            a_c = a_c + u * att_buf[cb + k, :]
                        return a_c
                    acc = lax.fori_loop(0, (ch // LANES) * TPB, acc_body,
                                        jnp.zeros((LANES,), jnp.float32))
                    pe.append(jnp.exp(acc))
                for h in range(heads):
                    hv = jnp.full((LANES,), h, jnp.int32)
                    plsc.store_scatter(den_buf, [rvec, hv], pe[h])
                for h in range(heads):
                    def scale_body(jj, cc, h=h):
                        b16 = jj // TPB
                        kk = jj - b16 * TPB
                        cb = h * ch + b16 * LANES
                        for k2 in range(U):
                            k = kk * U + k2
                            cv = cb + ((lane + k) & (LANES - 1))
                            a = plsc.load_gather(xl_buf, [rvec, cv])
                            plsc.store_scatter(xl_buf, [rvec, cv],
                                               a * pe[h])
                        return cc
                    lax.fori_loop(0, (ch // LANES) * TPB, scale_body, 0)
                return carry
            return group_body

        groups = (make_group(xlb0, xrb0, denb0), make_group(xlb1, xrb1, denb1))

        issue_idx(0, 0)
        wait_idx(0)
        issue_gather(0)
        issue_idx(1, 1)

        def pair_body(go, carry):
            for b in (0, 1):
                g = 2 * go + b
                nb = 1 - b
                wait_gather(b)

                @pl.when((g + 1 < nch) & (g >= 1))
                def _():
                    wait_scatter(nb)

                @pl.when(g + 1 < nch)
                def _():
                    wait_idx(nb)
                    issue_gather(nb)
                for i in range(chunk // LANES):
                    sdx[b][pl.ds(i * LANES, LANES)] = (
                        eib[b][1, pl.ds(i * LANES, LANES)])

                @pl.when(g + 2 < nch)
                def _():
                    issue_idx(g + 2, b)
                lax.fori_loop(0, chunk // LANES, groups[b], 0)
                issue_scatter(b)
            return carry

        lax.fori_loop(0, nch // 2, pair_body, 0)
        wait_scatter(0)
        wait_scatter(1)
        plsc.subcore_barrier()
        pltpu.sync_copy(out_sh.at[pl.ds(rbase, rpt)],
                        out_hbm.at[c, pl.ds(rbase, rpt)])
        pltpu.sync_copy(den_sh.at[pl.ds(rbase, rpt)],
                        den_hbm.at[c, pl.ds(rbase, rpt)])

    dbuf = [pltpu.VMEM((2, chunk), jnp.int32),
            pltpu.VMEM((chunk, f), jnp.float32),
            pltpu.VMEM((chunk, f), jnp.float32),
            pltpu.VMEM((chunk, DPAD), jnp.float32),
            pltpu.VMEM((chunk,), jnp.int32)]
    run = pl.kernel(
        body,
        out_type=[jax.ShapeDtypeStruct((NC, ndp, f), jnp.float32),
                  jax.ShapeDtypeStruct((NC, ndp, DPAD), jnp.float32)],
        mesh=mesh,
        compiler_params=pltpu.CompilerParams(needs_layout_passes=False,
                                             use_tc_tiling_on_sc=False),
        scratch_types=dbuf + dbuf + [
            pltpu.VMEM((f, LANES), jnp.float32),
            pltpu.VMEM_SHARED((ndp, f), jnp.float32),
            pltpu.VMEM_SHARED((ndp, DPAD), jnp.float32),
        ] + [pltpu.SemaphoreType.DMA] * 10,
    )
    rows = jnp.arange(f)
    cols = (rows // LANES * LANES)[:, None] + (
        (rows % LANES)[:, None] + jnp.arange(LANES)[None, :]) % LANES
    att_rot = att.reshape(-1)[cols]
    ei = jnp.stack([src, dst])
    op, dp = run(xl, xr, ei, att_rot)
    return op[:, :n_dst], dp[:, :n_dst]


def kernel(x_user, x_item, edge_index_user_to_item, edge_index_item_rev_user,
           Wl0_u2i, Wr0_u2i, att0_u2i, b0_u2i,
           Wl0_i2u, Wr0_i2u, att0_i2u, b0_i2u,
           Wl1_u2i, Wr1_u2i, att1_u2i, b1_u2i,
           Wl1_i2u, Wr1_i2u, att1_i2u, b1_i2u):
    n_user = x_user.shape[0]
    n_item = x_item.shape[0]
    heads0, ch0 = att0_u2i.shape
    out1 = att1_u2i.shape[1]

    ne = edge_index_user_to_item.shape[1]
    nep = -(-ne // (NC * NS * 320)) * (NC * NS * 320)
    pad_s = jnp.zeros((nep - ne,), edge_index_user_to_item.dtype)
    pad_d = jnp.full((nep - ne,), n_item, edge_index_user_to_item.dtype)
    src_u2i = jnp.concatenate([edge_index_user_to_item[0], pad_s])
    dst_u2i = jnp.concatenate([edge_index_user_to_item[1], pad_d])
    src_i2u = jnp.concatenate([edge_index_item_rev_user[0], pad_s])
    dst_i2u = jnp.concatenate([edge_index_item_rev_user[1],
                               jnp.full((nep - ne,), n_user,
                                        edge_index_item_rev_user.dtype)])

    xl0_u2i, xr0_i2u = _proj2(x_user, Wl0_u2i, Wr0_i2u)
    xl0_i2u, xr0_u2i = _proj2(x_item, Wl0_i2u, Wr0_u2i)

    op_i0, dp_i0 = _edge_sc(xl0_u2i, xr0_u2i, src_u2i, dst_u2i,
                            att0_u2i.reshape(-1), n_item, heads0, ch0)
    op_u0, dp_u0 = _edge_sc(xl0_i2u, xr0_i2u, src_i2u, dst_i2u,
                            att0_i2u.reshape(-1), n_user, heads0, ch0)

    xl1_u2i, xr1_i2u = _combo(op_u0, dp_u0, b0_i2u, Wl1_u2i, Wr1_i2u,
                              heads0, ch0)
    xl1_i2u, xr1_u2i = _combo(op_i0, dp_i0, b0_u2i, Wl1_i2u, Wr1_u2i,
                              heads0, ch0)

    op_i1, dp_i1 = _edge_sc(xl1_u2i, xr1_u2i, src_u2i, dst_u2i,
                            att1_u2i.reshape(-1), n_item, 1, out1)
    op_u1, dp_u1 = _edge_sc(xl1_i2u, xr1_i2u, src_i2u, dst_i2u,
                            att1_i2u.reshape(-1), n_user, 1, out1)

    out_user = _final(op_u1, dp_u1, b1_i2u, 1, out1)
    out_item = _final(op_i1, dp_i1, b1_u2i, 1, out1)
    return (out_user, out_item)
